# baseline (speedup 1.0000x reference)
import functools
import jax
import jax.numpy as jnp
from jax import lax
from jax.experimental import pallas as pl
from jax.experimental.pallas import tpu as pltpu, tpu_sc as plsc

_STRIDE = 4
_RADIUS = 0.2
_K = 32
_DPAD = 128


def _fps_body(pts_ref, idx_ref, cx_ref, cy_ref, cz_ref):
    b, _, n = pts_ref.shape
    m = idx_ref.shape[1]
    xs = pts_ref[:, 0, :]
    ys = pts_ref[:, 1, :]
    zs = pts_ref[:, 2, :]
    lane_n = lax.broadcasted_iota(jnp.int32, (b, n), 1)
    lane_128 = lax.broadcasted_iota(jnp.int32, (b, 128), 1)

    def body(t, carry):
        dist, last, bidx, bcx, bcy, bcz = carry
        eq = lane_n == last
        lx = jnp.sum(jnp.where(eq, xs, 0.0), axis=1, keepdims=True)
        ly = jnp.sum(jnp.where(eq, ys, 0.0), axis=1, keepdims=True)
        lz = jnp.sum(jnp.where(eq, zs, 0.0), axis=1, keepdims=True)
        tm = jnp.bitwise_and(t, 127)
        sel = lane_128 == tm
        bidx = jnp.where(sel, last, bidx)
        bcx = jnp.where(sel, lx, bcx)
        bcy = jnp.where(sel, ly, bcy)
        bcz = jnp.where(sel, lz, bcz)

        @pl.when(tm == 127)
        def _():
            base = pl.multiple_of(t - 127, 128)
            idx_ref[:, pl.ds(base, 128)] = bidx
            cx_ref[:, pl.ds(base, 128)] = bcx
            cy_ref[:, pl.ds(base, 128)] = bcy
            cz_ref[:, pl.ds(base, 128)] = bcz

        dx = xs - lx
        dy = ys - ly
        dz = zs - lz
        d = dx * dx + dy * dy
        d = d + dz * dz
        dist = jnp.minimum(dist, d)
        mx = jnp.max(dist, axis=1, keepdims=True)
        nxt = jnp.min(jnp.where(dist == mx, lane_n, n), axis=1, keepdims=True)
        return dist, nxt.astype(jnp.int32), bidx, bcx, bcy, bcz

    dist0 = jnp.full((b, n), 1e10, dtype=jnp.float32)
    last0 = jnp.zeros((b, 1), dtype=jnp.int32)
    buf_f = xs[:, :128]
    buf_i = buf_f.astype(jnp.int32)
    lax.fori_loop(0, m, body, (dist0, last0, buf_i, buf_f, buf_f, buf_f))


def _run_fps(pts_t, m):
    b, _, n = pts_t.shape
    return pl.pallas_call(
        _fps_body,
        out_shape=(
            jax.ShapeDtypeStruct((b, m), jnp.int32),
            jax.ShapeDtypeStruct((b, m), jnp.float32),
            jax.ShapeDtypeStruct((b, m), jnp.float32),
            jax.ShapeDtypeStruct((b, m), jnp.float32),
        ),
    )(pts_t)


def _ballq_body(pts_ref, cent_ref, gidx_ref, *, n, k):
    bi = pl.program_id(0)
    mt = cent_ref.shape[1]
    px = pts_ref[0, 0, :].reshape(1, n)
    py = pts_ref[0, 1, :].reshape(1, n)
    pz = pts_ref[0, 2, :].reshape(1, n)
    c = cent_ref[0]
    cx = c[:, 0:1]
    cy = c[:, 1:2]
    cz = c[:, 2:3]
    dx = cx - px
    dy = cy - py
    dz = cz - pz
    sqd = dx * dx + dy * dy
    sqd = sqd + dz * dz
    hits = sqd <= _RADIUS ** 2
    lane = lax.broadcasted_iota(jnp.int32, (mt, n), 1)
    big = jnp.int32(n)
    nch = n // 128
    hf = hits.astype(jnp.float32).reshape(mt * nch, 128)
    r0 = lax.broadcasted_iota(jnp.int32, (128, 128), 0)
    c0 = lax.broadcasted_iota(jnp.int32, (128, 128), 1)
    ltri = (r0 <= c0).astype(jnp.float32)
    cs = jnp.dot(hf, ltri, preferred_element_type=jnp.float32)
    cs = cs.reshape(mt, nch, 128)
    base = cs[:, :, 127]
    r1 = lax.broadcasted_iota(jnp.int32, (nch, nch), 0)
    c1 = lax.broadcasted_iota(jnp.int32, (nch, nch), 1)
    utri = (r1 < c1).astype(jnp.float32)
    exb = jnp.dot(base, utri, preferred_element_type=jnp.float32)
    rank = (cs + exb.reshape(mt, nch, 1)).reshape(mt, n)
    hitsf = hf.reshape(mt, n)
    rankv = rank * hitsf
    cols = []
    for kk in range(k):
        cand = jnp.min(jnp.where(rankv == float(kk + 1), lane, big),
                       axis=1, keepdims=True)
        cols.append(cand)
    idx = jnp.concatenate(cols, axis=1)
    first = idx[:, 0:1]
    idx = jnp.where(idx == big, first, idx)
    idx = jnp.where(idx == big, 0, idx)
    gidx_ref[0] = idx + bi * n


def _run_ballq(pts_t, centroids, m, mt, k):
    b, _, n = pts_t.shape
    return pl.pallas_call(
        functools.partial(_ballq_body, n=n, k=k),
        grid=(b, m // mt),
        in_specs=[
            pl.BlockSpec((1, 3, n), lambda i, j: (i, 0, 0)),
            pl.BlockSpec((1, mt, 3), lambda i, j: (i, j, 0)),
        ],
        out_specs=pl.BlockSpec((1, mt, k), lambda i, j: (i, j, 0)),
        out_shape=jax.ShapeDtypeStruct((b, m, k), jnp.int32),
    )(pts_t, centroids)


def _run_sc_gather(table, gidx_flat):
    rows_total, d = table.shape
    ntotal = gidx_flat.shape[0]
    info = plsc.get_sparse_core_info()
    nc, ns = info.num_cores, info.num_subcores
    nw = nc * ns
    per_w = ntotal // nw
    chunk = 128
    nchunks = per_w // chunk
    idx2d = gidx_flat.reshape(ntotal // chunk, chunk)
    mesh = plsc.VectorSubcoreMesh(core_axis_name="c", subcore_axis_name="s")

    @functools.partial(
        pl.kernel,
        mesh=mesh,
        out_type=jax.ShapeDtypeStruct((ntotal, d), jnp.float32),
        scratch_types=[
            pltpu.VMEM((nchunks, chunk), jnp.int32),
            pltpu.VMEM((chunk, d), jnp.float32),
            pltpu.SemaphoreType.DMA,
        ],
    )
    def gather_k(table_hbm, idx_hbm, out_hbm, idx_v, rows_v, sem):
        wid = lax.axis_index("s") * nc + lax.axis_index("c")
        row0 = wid * nchunks
        pltpu.sync_copy(idx_hbm.at[pl.ds(row0, nchunks)], idx_v)

        def body(i, _):
            pltpu.async_copy(table_hbm.at[idx_v.at[i]], rows_v, sem).wait()
            pltpu.sync_copy(rows_v, out_hbm.at[pl.ds((row0 + i) * chunk, chunk)])
            return 0

        lax.fori_loop(0, nchunks, body, 0)

    return gather_k(table, idx2d)


def _conv_body(g_ref, cent_ref, wt_ref, wc_ref, pmax_ref, pmin_ref, s_ref, q_ref, *, k):
    i = pl.program_id(0)
    j = pl.program_id(1)
    mt = g_ref.shape[1]
    d = g_ref.shape[3]
    oc = wt_ref.shape[1]
    rows = g_ref[0].reshape(mt * k, d)
    a = jnp.dot(rows, wt_ref[...], preferred_element_type=jnp.float32)
    a3 = a.reshape(mt, k, oc)
    amax = jnp.max(a3, axis=1)
    amin = jnp.min(a3, axis=1)
    asum = jnp.sum(a3, axis=1)
    asq = jnp.sum(a3 * a3, axis=1)
    c = cent_ref[0]
    cc = (c[:, 0:1] * wc_ref[0:1, :]
          + c[:, 1:2] * wc_ref[1:2, :]
          + c[:, 2:3] * wc_ref[2:3, :])
    pmax_ref[0] = amax - cc
    pmin_ref[0] = amin - cc
    part_s = jnp.sum(asum - k * cc, axis=0, keepdims=True)
    part_q = jnp.sum(asq - 2.0 * cc * asum + k * (cc * cc), axis=0, keepdims=True)

    @pl.when(jnp.logical_and(i == 0, j == 0))
    def _():
        s_ref[...] = jnp.zeros_like(s_ref)
        q_ref[...] = jnp.zeros_like(q_ref)

    s_ref[0:1, :] = s_ref[0:1, :] + part_s
    q_ref[0:1, :] = q_ref[0:1, :] + part_q


def _run_conv(gathered4, centroids, wt, wc, mt, k):
    b, m, _, d = gathered4.shape
    oc = wt.shape[1]
    return pl.pallas_call(
        functools.partial(_conv_body, k=k),
        grid=(b, m // mt),
        in_specs=[
            pl.BlockSpec((1, mt, k, d), lambda i, j: (i, j, 0, 0)),
            pl.BlockSpec((1, mt, 3), lambda i, j: (i, j, 0)),
            pl.BlockSpec((d, oc), lambda i, j: (0, 0)),
            pl.BlockSpec((8, oc), lambda i, j: (0, 0)),
        ],
        out_specs=(
            pl.BlockSpec((1, mt, oc), lambda i, j: (i, j, 0)),
            pl.BlockSpec((1, mt, oc), lambda i, j: (i, j, 0)),
            pl.BlockSpec((8, oc), lambda i, j: (0, 0)),
            pl.BlockSpec((8, oc), lambda i, j: (0, 0)),
        ),
        out_shape=(
            jax.ShapeDtypeStruct((b, m, oc), jnp.float32),
            jax.ShapeDtypeStruct((b, m, oc), jnp.float32),
            jax.ShapeDtypeStruct((8, oc), jnp.float32),
            jax.ShapeDtypeStruct((8, oc), jnp.float32),
        ),
    )(gathered4, centroids, wt, wc)


def _bn_body(pmax_ref, pmin_ref, s_ref, q_ref, gam_ref, bet_ref, out_ref, *, count):
    s = s_ref[0:1, :]
    q = q_ref[0:1, :]
    mean = s / count
    var = q / count - mean * mean
    rstd = lax.rsqrt(var + 1e-5)
    scale = gam_ref[0:1, :] * rstd
    shift = bet_ref[0:1, :] - mean * scale
    pooled = jnp.where(scale >= 0.0, pmax_ref[0], pmin_ref[0])
    out_ref[0] = jnp.maximum(pooled * scale + shift, 0.0)


def _run_bn(pmax, pmin, s, q, gam, bet, count):
    b, m, oc = pmax.shape
    return pl.pallas_call(
        functools.partial(_bn_body, count=count),
        grid=(b,),
        in_specs=[
            pl.BlockSpec((1, m, oc), lambda i: (i, 0, 0)),
            pl.BlockSpec((1, m, oc), lambda i: (i, 0, 0)),
            pl.BlockSpec((8, oc), lambda i: (0, 0)),
            pl.BlockSpec((8, oc), lambda i: (0, 0)),
            pl.BlockSpec((8, oc), lambda i: (0, 0)),
            pl.BlockSpec((8, oc), lambda i: (0, 0)),
        ],
        out_specs=pl.BlockSpec((1, m, oc), lambda i: (i, 0, 0)),
        out_shape=jax.ShapeDtypeStruct((b, m, oc), jnp.float32),
    )(pmax, pmin, s, q, gam, bet)


@jax.jit
def kernel(points, features, W, gamma, beta):
    b, n, _ = points.shape
    c_in = features.shape[-1]
    oc = W.shape[0]
    m = n // _STRIDE
    k = _K

    pts_t = jnp.transpose(points, (0, 2, 1))

    _, cx, cy, cz = _run_fps(pts_t, m)
    centroids = jnp.stack([cx, cy, cz], axis=-1)

    gidx = _run_ballq(pts_t, centroids, m, 256, k)

    table = jnp.concatenate([points, features], axis=-1)
    table = jnp.pad(table, ((0, 0), (0, 0), (0, _DPAD - 3 - c_in)))
    table = table.reshape(b * n, _DPAD)
    gathered = _run_sc_gather(table, gidx.reshape(b * m * k))
    gathered4 = gathered.reshape(b, m, k, _DPAD)

    wp = jnp.transpose(W[:, :3], (1, 0)) / _RADIUS
    wf = jnp.transpose(W[:, 3:], (1, 0))
    wt = jnp.concatenate(
        [wp, wf, jnp.zeros((_DPAD - 3 - c_in, oc), jnp.float32)], axis=0)
    wc = jnp.concatenate([wp, jnp.zeros((5, oc), jnp.float32)], axis=0)

    pmax, pmin, ssum, ssq = _run_conv(gathered4, centroids, wt, wc, 128, k)

    gam = jnp.concatenate(
        [gamma.reshape(1, oc), jnp.zeros((7, oc), jnp.float32)], axis=0)
    bet = jnp.concatenate(
        [beta.reshape(1, oc), jnp.zeros((7, oc), jnp.float32)], axis=0)
    group_features = _run_bn(pmax, pmin, ssum, ssq, gam, bet, float(b * k * m))

    return (centroids, group_features)


# speedup vs baseline: 1.0119x; 1.0119x over previous
import functools
import jax
import jax.numpy as jnp
from jax import lax
from jax.experimental import pallas as pl
from jax.experimental.pallas import tpu as pltpu, tpu_sc as plsc

_STRIDE = 4
_RADIUS = 0.2
_K = 32
_DPAD = 128


def _fps_body(pts_ref, idx_ref, cx_ref, cy_ref, cz_ref):
    b, _, n = pts_ref.shape
    m = idx_ref.shape[1]
    xs = pts_ref[:, 0, :]
    ys = pts_ref[:, 1, :]
    zs = pts_ref[:, 2, :]
    lane_n = lax.broadcasted_iota(jnp.int32, (b, n), 1)
    lane_m = lax.broadcasted_iota(jnp.int32, (b, m), 1)

    def body(t, carry):
        dist, last = carry
        eq = lane_n == last
        lx = jnp.sum(jnp.where(eq, xs, 0.0), axis=1, keepdims=True)
        ly = jnp.sum(jnp.where(eq, ys, 0.0), axis=1, keepdims=True)
        lz = jnp.sum(jnp.where(eq, zs, 0.0), axis=1, keepdims=True)
        mask_t = lane_m == t
        idx_ref[...] = jnp.where(mask_t, last, idx_ref[...])
        cx_ref[...] = jnp.where(mask_t, lx, cx_ref[...])
        cy_ref[...] = jnp.where(mask_t, ly, cy_ref[...])
        cz_ref[...] = jnp.where(mask_t, lz, cz_ref[...])
        dx = xs - lx
        dy = ys - ly
        dz = zs - lz
        d = dx * dx + dy * dy
        d = d + dz * dz
        dist = jnp.minimum(dist, d)
        mx = jnp.max(dist, axis=1, keepdims=True)
        nxt = jnp.min(jnp.where(dist == mx, lane_n, n), axis=1, keepdims=True)
        return dist, nxt.astype(jnp.int32)

    dist0 = jnp.full((b, n), 1e10, dtype=jnp.float32)
    last0 = jnp.zeros((b, 1), dtype=jnp.int32)
    lax.fori_loop(0, m, body, (dist0, last0))


def _run_fps(pts_t, m):
    b, _, n = pts_t.shape
    return pl.pallas_call(
        _fps_body,
        out_shape=(
            jax.ShapeDtypeStruct((b, m), jnp.int32),
            jax.ShapeDtypeStruct((b, m), jnp.float32),
            jax.ShapeDtypeStruct((b, m), jnp.float32),
            jax.ShapeDtypeStruct((b, m), jnp.float32),
        ),
    )(pts_t)


def _ballq_body(pts_ref, cent_ref, gidx_ref, *, n, k):
    bi = pl.program_id(0)
    mt = cent_ref.shape[1]
    px = pts_ref[0, 0, :].reshape(1, n)
    py = pts_ref[0, 1, :].reshape(1, n)
    pz = pts_ref[0, 2, :].reshape(1, n)
    c = cent_ref[0]
    cx = c[:, 0:1]
    cy = c[:, 1:2]
    cz = c[:, 2:3]
    dx = cx - px
    dy = cy - py
    dz = cz - pz
    sqd = dx * dx + dy * dy
    sqd = sqd + dz * dz
    hits = sqd <= _RADIUS ** 2
    lane = lax.broadcasted_iota(jnp.int32, (mt, n), 1)
    big = jnp.int32(n)
    nch = n // 128
    hf = hits.astype(jnp.float32).reshape(mt * nch, 128)
    r0 = lax.broadcasted_iota(jnp.int32, (128, 128), 0)
    c0 = lax.broadcasted_iota(jnp.int32, (128, 128), 1)
    ltri = (r0 <= c0).astype(jnp.float32)
    cs = jnp.dot(hf, ltri, preferred_element_type=jnp.float32)
    cs = cs.reshape(mt, nch, 128)
    base = cs[:, :, 127]
    r1 = lax.broadcasted_iota(jnp.int32, (nch, nch), 0)
    c1 = lax.broadcasted_iota(jnp.int32, (nch, nch), 1)
    utri = (r1 < c1).astype(jnp.float32)
    exb = jnp.dot(base, utri, preferred_element_type=jnp.float32)
    rank = (cs + exb.reshape(mt, nch, 1)).reshape(mt, n)
    hitsf = hf.reshape(mt, n)
    rankv = rank * hitsf
    cols = []
    for kk in range(k):
        cand = jnp.min(jnp.where(rankv == float(kk + 1), lane, big),
                       axis=1, keepdims=True)
        cols.append(cand)
    idx = jnp.concatenate(cols, axis=1)
    first = idx[:, 0:1]
    idx = jnp.where(idx == big, first, idx)
    idx = jnp.where(idx == big, 0, idx)
    gidx_ref[0] = idx + bi * n


def _run_ballq(pts_t, centroids, m, mt, k):
    b, _, n = pts_t.shape
    return pl.pallas_call(
        functools.partial(_ballq_body, n=n, k=k),
        grid=(b, m // mt),
        in_specs=[
            pl.BlockSpec((1, 3, n), lambda i, j: (i, 0, 0)),
            pl.BlockSpec((1, mt, 3), lambda i, j: (i, j, 0)),
        ],
        out_specs=pl.BlockSpec((1, mt, k), lambda i, j: (i, j, 0)),
        out_shape=jax.ShapeDtypeStruct((b, m, k), jnp.int32),
    )(pts_t, centroids)


def _run_sc_gather(table, gidx_flat):
    rows_total, d = table.shape
    ntotal = gidx_flat.shape[0]
    info = plsc.get_sparse_core_info()
    nc, ns = info.num_cores, info.num_subcores
    nw = nc * ns
    per_w = ntotal // nw
    chunk = 128
    nchunks = per_w // chunk
    idx2d = gidx_flat.reshape(ntotal // chunk, chunk)
    mesh = plsc.VectorSubcoreMesh(core_axis_name="c", subcore_axis_name="s")

    @functools.partial(
        pl.kernel,
        mesh=mesh,
        out_type=jax.ShapeDtypeStruct((ntotal, d), jnp.float32),
        scratch_types=[
            pltpu.VMEM((nchunks, chunk), jnp.int32),
            pltpu.VMEM((chunk, d), jnp.float32),
            pltpu.SemaphoreType.DMA,
        ],
    )
    def gather_k(table_hbm, idx_hbm, out_hbm, idx_v, rows_v, sem):
        wid = lax.axis_index("s") * nc + lax.axis_index("c")
        row0 = wid * nchunks
        pltpu.sync_copy(idx_hbm.at[pl.ds(row0, nchunks)], idx_v)

        def body(i, _):
            pltpu.async_copy(table_hbm.at[idx_v.at[i]], rows_v, sem).wait()
            pltpu.sync_copy(rows_v, out_hbm.at[pl.ds((row0 + i) * chunk, chunk)])
            return 0

        lax.fori_loop(0, nchunks, body, 0)

    return gather_k(table, idx2d)


def _conv_body(g_ref, cent_ref, wt_ref, wc_ref, pmax_ref, pmin_ref, s_ref, q_ref, *, k):
    i = pl.program_id(0)
    j = pl.program_id(1)
    mt = g_ref.shape[1]
    d = g_ref.shape[3]
    oc = wt_ref.shape[1]
    rows = g_ref[0].reshape(mt * k, d)
    a = jnp.dot(rows, wt_ref[...], preferred_element_type=jnp.float32)
    a3 = a.reshape(mt, k, oc)
    amax = jnp.max(a3, axis=1)
    amin = jnp.min(a3, axis=1)
    asum = jnp.sum(a3, axis=1)
    asq = jnp.sum(a3 * a3, axis=1)
    c = cent_ref[0]
    cc = (c[:, 0:1] * wc_ref[0:1, :]
          + c[:, 1:2] * wc_ref[1:2, :]
          + c[:, 2:3] * wc_ref[2:3, :])
    pmax_ref[0] = amax - cc
    pmin_ref[0] = amin - cc
    part_s = jnp.sum(asum - k * cc, axis=0, keepdims=True)
    part_q = jnp.sum(asq - 2.0 * cc * asum + k * (cc * cc), axis=0, keepdims=True)

    @pl.when(jnp.logical_and(i == 0, j == 0))
    def _():
        s_ref[...] = jnp.zeros_like(s_ref)
        q_ref[...] = jnp.zeros_like(q_ref)

    s_ref[0:1, :] = s_ref[0:1, :] + part_s
    q_ref[0:1, :] = q_ref[0:1, :] + part_q


def _run_conv(gathered4, centroids, wt, wc, mt, k):
    b, m, _, d = gathered4.shape
    oc = wt.shape[1]
    return pl.pallas_call(
        functools.partial(_conv_body, k=k),
        grid=(b, m // mt),
        in_specs=[
            pl.BlockSpec((1, mt, k, d), lambda i, j: (i, j, 0, 0)),
            pl.BlockSpec((1, mt, 3), lambda i, j: (i, j, 0)),
            pl.BlockSpec((d, oc), lambda i, j: (0, 0)),
            pl.BlockSpec((8, oc), lambda i, j: (0, 0)),
        ],
        out_specs=(
            pl.BlockSpec((1, mt, oc), lambda i, j: (i, j, 0)),
            pl.BlockSpec((1, mt, oc), lambda i, j: (i, j, 0)),
            pl.BlockSpec((8, oc), lambda i, j: (0, 0)),
            pl.BlockSpec((8, oc), lambda i, j: (0, 0)),
        ),
        out_shape=(
            jax.ShapeDtypeStruct((b, m, oc), jnp.float32),
            jax.ShapeDtypeStruct((b, m, oc), jnp.float32),
            jax.ShapeDtypeStruct((8, oc), jnp.float32),
            jax.ShapeDtypeStruct((8, oc), jnp.float32),
        ),
    )(gathered4, centroids, wt, wc)


def _bn_body(pmax_ref, pmin_ref, s_ref, q_ref, gam_ref, bet_ref, out_ref, *, count):
    s = s_ref[0:1, :]
    q = q_ref[0:1, :]
    mean = s / count
    var = q / count - mean * mean
    rstd = lax.rsqrt(var + 1e-5)
    scale = gam_ref[0:1, :] * rstd
    shift = bet_ref[0:1, :] - mean * scale
    pooled = jnp.where(scale >= 0.0, pmax_ref[0], pmin_ref[0])
    out_ref[0] = jnp.maximum(pooled * scale + shift, 0.0)


def _run_bn(pmax, pmin, s, q, gam, bet, count):
    b, m, oc = pmax.shape
    return pl.pallas_call(
        functools.partial(_bn_body, count=count),
        grid=(b,),
        in_specs=[
            pl.BlockSpec((1, m, oc), lambda i: (i, 0, 0)),
            pl.BlockSpec((1, m, oc), lambda i: (i, 0, 0)),
            pl.BlockSpec((8, oc), lambda i: (0, 0)),
            pl.BlockSpec((8, oc), lambda i: (0, 0)),
            pl.BlockSpec((8, oc), lambda i: (0, 0)),
            pl.BlockSpec((8, oc), lambda i: (0, 0)),
        ],
        out_specs=pl.BlockSpec((1, m, oc), lambda i: (i, 0, 0)),
        out_shape=jax.ShapeDtypeStruct((b, m, oc), jnp.float32),
    )(pmax, pmin, s, q, gam, bet)


@jax.jit
def kernel(points, features, W, gamma, beta):
    b, n, _ = points.shape
    c_in = features.shape[-1]
    oc = W.shape[0]
    m = n // _STRIDE
    k = _K

    pts_t = jnp.transpose(points, (0, 2, 1))

    _, cx, cy, cz = _run_fps(pts_t, m)
    centroids = jnp.stack([cx, cy, cz], axis=-1)

    gidx = _run_ballq(pts_t, centroids, m, 128, k)

    table = jnp.concatenate([points, features], axis=-1)
    table = jnp.pad(table, ((0, 0), (0, 0), (0, _DPAD - 3 - c_in)))
    table = table.reshape(b * n, _DPAD)
    gathered = _run_sc_gather(table, gidx.reshape(b * m * k))
    gathered4 = gathered.reshape(b, m, k, _DPAD)

    wp = jnp.transpose(W[:, :3], (1, 0)) / _RADIUS
    wf = jnp.transpose(W[:, 3:], (1, 0))
    wt = jnp.concatenate(
        [wp, wf, jnp.zeros((_DPAD - 3 - c_in, oc), jnp.float32)], axis=0)
    wc = jnp.concatenate([wp, jnp.zeros((5, oc), jnp.float32)], axis=0)

    pmax, pmin, ssum, ssq = _run_conv(gathered4, centroids, wt, wc, 128, k)

    gam = jnp.concatenate(
        [gamma.reshape(1, oc), jnp.zeros((7, oc), jnp.float32)], axis=0)
    bet = jnp.concatenate(
        [beta.reshape(1, oc), jnp.zeros((7, oc), jnp.float32)], axis=0)
    group_features = _run_bn(pmax, pmin, ssum, ssq, gam, bet, float(b * k * m))

    return (centroids, group_features)
